# Initial kernel scaffold; baseline (speedup 1.0000x reference)
import jax
import jax.numpy as jnp
from jax.experimental import pallas as pl


def kernel(x, delays):
    raise NotImplementedError("write your pallas kernel here")


# speedup vs baseline: 17.4185x; 17.4185x over previous
import jax
import jax.numpy as jnp
from jax.experimental import pallas as pl
from jax.experimental.pallas import tpu as pltpu

DMAX = 16
CB = 128


def _shift_kernel(d_ref, x_ref, o_ref):
    x = x_ref[0]
    T = x.shape[0]
    d = d_ref[...]
    z = jnp.pad(x, ((DMAX, DMAX), (0, 0)))
    w = z
    for k in range(5):
        s = 1 << k
        mask = ((d >> k) & 1) == 1
        shifted = jnp.pad(w, ((s, 0), (0, 0)))[:-s]
        w = jnp.where(mask, shifted, w)
    o_ref[0] = w[DMAX:]


def kernel(x, delays):
    B, T, C = x.shape
    Tp = T + DMAX
    d2 = delays.astype(jnp.int32).reshape(1, C)
    grid = (B, C // CB)
    return pl.pallas_call(
        _shift_kernel,
        grid=grid,
        in_specs=[
            pl.BlockSpec((1, CB), lambda b, c: (0, c)),
            pl.BlockSpec((1, T, CB), lambda b, c: (b, 0, c)),
        ],
        out_specs=pl.BlockSpec((1, Tp, CB), lambda b, c: (b, 0, c)),
        out_shape=jax.ShapeDtypeStruct((B, Tp, C), x.dtype),
        compiler_params=pltpu.CompilerParams(
            dimension_semantics=("parallel", "parallel"),
        ),
    )(d2, x)


# speedup vs baseline: 20.0873x; 1.1532x over previous
import jax
import jax.numpy as jnp
from jax.experimental import pallas as pl
from jax.experimental.pallas import tpu as pltpu

DMAX = 16
CB = 512


def _shift_kernel(d_ref, x_ref, o_ref):
    x = x_ref[0]
    T = x.shape[0]
    d = d_ref[...]
    z = jnp.pad(x, ((DMAX, DMAX), (0, 0)))
    w = z
    for k in range(5):
        s = 1 << k
        mask = ((d >> k) & 1) == 1
        shifted = jnp.pad(w, ((s, 0), (0, 0)))[:-s]
        w = jnp.where(mask, shifted, w)
    o_ref[0] = w[DMAX:]


def kernel(x, delays):
    B, T, C = x.shape
    Tp = T + DMAX
    d2 = delays.astype(jnp.int32).reshape(1, C)
    grid = (B, C // CB)
    return pl.pallas_call(
        _shift_kernel,
        grid=grid,
        in_specs=[
            pl.BlockSpec((1, CB), lambda b, c: (0, c)),
            pl.BlockSpec((1, T, CB), lambda b, c: (b, 0, c)),
        ],
        out_specs=pl.BlockSpec((1, Tp, CB), lambda b, c: (b, 0, c)),
        out_shape=jax.ShapeDtypeStruct((B, Tp, C), x.dtype),
        compiler_params=pltpu.CompilerParams(
            dimension_semantics=("parallel", "parallel"),
        ),
    )(d2, x)


# speedup vs baseline: 20.2423x; 1.0077x over previous
import jax
import jax.numpy as jnp
from jax.experimental import pallas as pl
from jax.experimental.pallas import tpu as pltpu

DMAX = 16
CB = 256


def _shift_kernel(d_ref, x_ref, o_ref):
    x = x_ref[0]
    T = x.shape[0]
    d = d_ref[...]
    z = jnp.pad(x, ((DMAX, DMAX), (0, 0)))
    w = z
    for k in range(5):
        s = 1 << k
        mask = ((d >> k) & 1) == 1
        shifted = jnp.pad(w, ((s, 0), (0, 0)))[:-s]
        w = jnp.where(mask, shifted, w)
    o_ref[0] = w[DMAX:]


def kernel(x, delays):
    B, T, C = x.shape
    Tp = T + DMAX
    d2 = delays.astype(jnp.int32).reshape(1, C)
    grid = (B, C // CB)
    return pl.pallas_call(
        _shift_kernel,
        grid=grid,
        in_specs=[
            pl.BlockSpec((1, CB), lambda b, c: (0, c)),
            pl.BlockSpec((1, T, CB), lambda b, c: (b, 0, c)),
        ],
        out_specs=pl.BlockSpec((1, Tp, CB), lambda b, c: (b, 0, c)),
        out_shape=jax.ShapeDtypeStruct((B, Tp, C), x.dtype),
        compiler_params=pltpu.CompilerParams(
            dimension_semantics=("parallel", "parallel"),
        ),
    )(d2, x)
